# baseline (speedup 1.0000x reference)
import functools

import jax
import jax.numpy as jnp
from jax import lax
from jax.experimental import pallas as pl
from jax.experimental.pallas import tpu as pltpu
from jax.experimental.pallas import tpu_sc as plsc

_CHUNK = 128


@functools.lru_cache(maxsize=None)
def _make_gather(V, D, B):
    info = plsc.get_sparse_core_info()
    nw = info.num_cores * info.num_subcores
    assert B % (8 * nw) == 0 and D % info.num_lanes == 0
    b_per_w = B // nw
    assert b_per_w % _CHUNK == 0
    n_chunks = b_per_w // _CHUNK
    mesh = plsc.VectorSubcoreMesh(core_axis_name="c", subcore_axis_name="s")

    @functools.partial(
        pl.kernel,
        mesh=mesh,
        out_type=jax.ShapeDtypeStruct((B, D), jnp.float32),
        compiler_params=pltpu.CompilerParams(use_tc_tiling_on_sc=False),
        scratch_types=[
            pltpu.VMEM((b_per_w,), jnp.int32),
            pltpu.VMEM((_CHUNK, D), jnp.float32),
            pltpu.SemaphoreType.DMA,
        ],
    )
    def gather_kernel(idx_hbm, table_hbm, out_hbm, idx_v, rows_v, sem):
        wid = lax.axis_index("s") * info.num_cores + lax.axis_index("c")
        base = wid * b_per_w
        pltpu.sync_copy(idx_hbm.at[pl.ds(base, b_per_w)], idx_v)

        def body(j, carry):
            idx_slice = idx_v.at[pl.ds(j * _CHUNK, _CHUNK)]
            pltpu.async_copy(table_hbm.at[idx_slice], rows_v, sem).wait()
            pltpu.sync_copy(rows_v, out_hbm.at[pl.ds(base + j * _CHUNK, _CHUNK)])
            return carry

        lax.fori_loop(0, n_chunks, body, 0)

    return gather_kernel


def kernel(ids, weight):
    bsz, hist = ids.shape
    vocab, dim = weight.shape
    flat = ids.reshape(bsz * hist)
    out = _make_gather(vocab, dim, bsz * hist)(flat, weight)
    return out.reshape(bsz, hist, dim)


# speedup vs baseline: 1.0603x; 1.0603x over previous
import functools

import jax
import jax.numpy as jnp
from jax import lax
from jax.experimental import pallas as pl
from jax.experimental.pallas import tpu as pltpu
from jax.experimental.pallas import tpu_sc as plsc

_CHUNK = 128
_STREAMS = 4
_GROUP = _CHUNK * _STREAMS


@functools.lru_cache(maxsize=None)
def _make_gather(V, D, B):
    info = plsc.get_sparse_core_info()
    nw = info.num_cores * info.num_subcores
    assert B % (8 * nw) == 0 and D % info.num_lanes == 0
    b_per_w = B // nw
    assert b_per_w % _GROUP == 0
    n_groups = b_per_w // _GROUP
    n_pairs = n_groups // 2
    assert n_pairs * 2 == n_groups and n_pairs >= 2
    mesh = plsc.VectorSubcoreMesh(core_axis_name="c", subcore_axis_name="s")

    @functools.partial(
        pl.kernel,
        mesh=mesh,
        out_type=jax.ShapeDtypeStruct((B, D), jnp.float32),
        compiler_params=pltpu.CompilerParams(use_tc_tiling_on_sc=False),
        scratch_types=[
            pltpu.VMEM((b_per_w,), jnp.int32),
            pltpu.VMEM((_GROUP, D), jnp.float32),
            pltpu.VMEM((_GROUP, D), jnp.float32),
            pltpu.SemaphoreType.DMA,
            pltpu.SemaphoreType.DMA,
            pltpu.SemaphoreType.DMA,
            pltpu.SemaphoreType.DMA,
        ],
    )
    def gather_kernel(idx_hbm, table_hbm, out_hbm, idx_v, buf0, buf1,
                      gsem0, gsem1, ssem0, ssem1):
        wid = lax.axis_index("s") * info.num_cores + lax.axis_index("c")
        base = wid * b_per_w
        pltpu.sync_copy(idx_hbm.at[pl.ds(base, b_per_w)], idx_v)

        def g_copy(g, buf, sem, k):
            idx_slice = idx_v.at[pl.ds(g * _GROUP + k * _CHUNK, _CHUNK)]
            return pltpu.make_async_copy(
                table_hbm.at[idx_slice], buf.at[pl.ds(k * _CHUNK, _CHUNK)], sem)

        def fire(g, buf, sem):
            for k in range(_STREAMS):
                g_copy(g, buf, sem, k).start()

        def wait_g(g, buf, sem):
            for k in range(_STREAMS):
                g_copy(g, buf, sem, k).wait()

        def s_copy(g, buf, sem):
            return pltpu.make_async_copy(
                buf, out_hbm.at[pl.ds(base + g * _GROUP, _GROUP)], sem)

        fire(0, buf0, gsem0)
        fire(1, buf1, gsem1)
        wait_g(0, buf0, gsem0)
        s_copy(0, buf0, ssem0).start()
        s_copy(0, buf0, ssem0).wait()
        fire(2, buf0, gsem0)
        wait_g(1, buf1, gsem1)
        s_copy(1, buf1, ssem1).start()

        def body(t, carry):
            a = 2 * t
            b = a + 1
            s_copy(b - 2, buf1, ssem1).wait()
            fire(b, buf1, gsem1)
            wait_g(a, buf0, gsem0)
            s_copy(a, buf0, ssem0).start()
            s_copy(a, buf0, ssem0).wait()
            fire(a + 2, buf0, gsem0)
            wait_g(b, buf1, gsem1)
            s_copy(b, buf1, ssem1).start()
            return carry

        lax.fori_loop(1, n_pairs - 1, body, 0)

        a = 2 * (n_pairs - 1)
        b = a + 1
        s_copy(b - 2, buf1, ssem1).wait()
        fire(b, buf1, gsem1)
        wait_g(a, buf0, gsem0)
        s_copy(a, buf0, ssem0).start()
        s_copy(a, buf0, ssem0).wait()
        wait_g(b, buf1, gsem1)
        s_copy(b, buf1, ssem1).start()
        s_copy(b, buf1, ssem1).wait()

    return gather_kernel


def kernel(ids, weight):
    bsz, hist = ids.shape
    vocab, dim = weight.shape
    flat = ids.reshape(bsz * hist)
    out = _make_gather(vocab, dim, bsz * hist)(flat, weight)
    return out.reshape(bsz, hist, dim)


# speedup vs baseline: 1.0617x; 1.0013x over previous
import functools

import jax
import jax.numpy as jnp
from jax import lax
from jax.experimental import pallas as pl
from jax.experimental.pallas import tpu as pltpu
from jax.experimental.pallas import tpu_sc as plsc

_GB = 32
_HPAD = 24


@functools.lru_cache(maxsize=None)
def _make_gather(V, D, B, H):
    info = plsc.get_sparse_core_info()
    nw = info.num_cores * info.num_subcores
    lanes = info.num_lanes
    assert D % lanes == 0 and B % nw == 0 and H <= _HPAD
    b_per_w = B // nw
    assert b_per_w % (_GB * 2) == 0 and b_per_w % lanes == 0
    n_groups = b_per_w // _GB
    n_pairs = n_groups // 2
    assert n_pairs >= 2
    mesh = plsc.VectorSubcoreMesh(core_axis_name="c", subcore_axis_name="s")

    @functools.partial(
        pl.kernel,
        mesh=mesh,
        out_type=jax.ShapeDtypeStruct((B, H, D), jnp.float32),
        compiler_params=pltpu.CompilerParams(
            use_tc_tiling_on_sc=False, needs_layout_passes=False),
        scratch_types=[
            pltpu.VMEM((H, b_per_w), jnp.int32),
            pltpu.VMEM((b_per_w * _HPAD,), jnp.int32),
            pltpu.VMEM((_GB, H, D), jnp.float32),
            pltpu.VMEM((_GB, H, D), jnp.float32),
            pltpu.SemaphoreType.DMA,
            pltpu.SemaphoreType.DMA,
            pltpu.SemaphoreType.DMA,
            pltpu.SemaphoreType.DMA,
        ],
    )
    def gather_kernel(ids_hbm, table_hbm, out_hbm, idx_hw, idx_bw, buf0, buf1,
                      gsem0, gsem1, ssem0, ssem1):
        wid = lax.axis_index("s") * info.num_cores + lax.axis_index("c")
        base = wid * b_per_w
        pltpu.sync_copy(ids_hbm.at[:, pl.ds(base, b_per_w)], idx_hw)

        lane = lax.iota(jnp.int32, lanes)

        def tr_h(h, carry):
            def tr_c(c, carry2):
                v = idx_hw[h, pl.ds(c * lanes, lanes)]
                addr = (c * lanes + lane) * _HPAD + h
                plsc.store_scatter(idx_bw, [addr], v)
                return carry2

            return lax.fori_loop(0, b_per_w // lanes, tr_c, carry)

        lax.fori_loop(0, H, tr_h, 0)

        def g_copy(l, buf, bb, sem):
            idx_slice = idx_bw.at[pl.ds(l * _HPAD, H)]
            return pltpu.make_async_copy(table_hbm.at[idx_slice], buf.at[bb], sem)

        def fire(g, buf, sem):
            def fb(bb, carry):
                g_copy(g * _GB + bb, buf, bb, sem).start()
                return carry

            lax.fori_loop(0, _GB, fb, 0)

        def wait_g(g, buf, sem):
            def wb(bb, carry):
                g_copy(g * _GB + bb, buf, bb, sem).wait()
                return carry

            lax.fori_loop(0, _GB, wb, 0)

        def s_copy(g, buf, sem):
            return pltpu.make_async_copy(
                buf, out_hbm.at[pl.ds(base + g * _GB, _GB)], sem)

        fire(0, buf0, gsem0)
        fire(1, buf1, gsem1)
        wait_g(0, buf0, gsem0)
        s_copy(0, buf0, ssem0).start()
        s_copy(0, buf0, ssem0).wait()
        fire(2, buf0, gsem0)
        wait_g(1, buf1, gsem1)
        s_copy(1, buf1, ssem1).start()

        def body(t, carry):
            a = 2 * t
            b = a + 1
            s_copy(b - 2, buf1, ssem1).wait()
            fire(b, buf1, gsem1)
            wait_g(a, buf0, gsem0)
            s_copy(a, buf0, ssem0).start()
            s_copy(a, buf0, ssem0).wait()
            fire(a + 2, buf0, gsem0)
            wait_g(b, buf1, gsem1)
            s_copy(b, buf1, ssem1).start()
            return carry

        lax.fori_loop(1, n_pairs - 1, body, 0)

        a = 2 * (n_pairs - 1)
        b = a + 1
        s_copy(b - 2, buf1, ssem1).wait()
        fire(b, buf1, gsem1)
        wait_g(a, buf0, gsem0)
        s_copy(a, buf0, ssem0).start()
        s_copy(a, buf0, ssem0).wait()
        wait_g(b, buf1, gsem1)
        s_copy(b, buf1, ssem1).start()
        s_copy(b, buf1, ssem1).wait()

    return gather_kernel


def kernel(ids, weight):
    bsz, hist = ids.shape
    vocab, dim = weight.shape
    ids_t = ids.T
    return _make_gather(vocab, dim, bsz, hist)(ids_t, weight)
